# Initial kernel scaffold; baseline (speedup 1.0000x reference)
import jax
import jax.numpy as jnp
from jax.experimental import pallas as pl


def kernel(P_data, A_data, q, b, x, y, s, P_rows, P_cols, A_rows, A_cols):
    raise NotImplementedError("write your pallas kernel here")


# speedup vs baseline: 330.1394x; 330.1394x over previous
import functools

import jax
import jax.numpy as jnp
from jax import lax
from jax.experimental import pallas as pl
from jax.experimental.pallas import tpu as pltpu
from jax.experimental.pallas import tpu_sc as plsc

NC = 2
NS = 16
NW = NC * NS
LANES = 16
CHUNK = 4096
VPC = CHUNK // LANES


def _sc_spmv_kernel(n, m, nnz):
    nnz_per_w = nnz // NW
    n_chunks = nnz_per_w // CHUNK
    assert nnz_per_w % CHUNK == 0

    mesh = plsc.VectorSubcoreMesh(core_axis_name="c", subcore_axis_name="s")
    f32 = jnp.float32

    @functools.partial(
        pl.kernel,
        out_type=(
            jax.ShapeDtypeStruct((NW, n), f32),
            jax.ShapeDtypeStruct((NW, n), f32),
            jax.ShapeDtypeStruct((NW, m), f32),
        ),
        mesh=mesh,
        compiler_params=pltpu.CompilerParams(needs_layout_passes=False),
        scratch_types=dict(
            x_v=pltpu.MemorySpace.VMEM((n,), f32),
            wy_v=pltpu.MemorySpace.VMEM((m,), f32),
            y_v=pltpu.MemorySpace.VMEM((m,), f32),
            s_v=pltpu.MemorySpace.VMEM((m,), f32),
            acc_px=pltpu.MemorySpace.VMEM((n,), f32),
            acc_aty=pltpu.MemorySpace.VMEM((n,), f32),
            acc_ax=pltpu.MemorySpace.VMEM((m,), f32),
            rows_v=pltpu.MemorySpace.VMEM((CHUNK,), jnp.int32),
            cols_v=pltpu.MemorySpace.VMEM((CHUNK,), jnp.int32),
            data_v=pltpu.MemorySpace.VMEM((CHUNK,), f32),
            sem0=pltpu.SemaphoreType.DMA,
            sem1=pltpu.SemaphoreType.DMA,
            sem2=pltpu.SemaphoreType.DMA,
        ),
    )
    def spmv(p_data, p_rows, p_cols, a_data, a_rows, a_cols, x_h, y_h, s_h,
             o_px, o_aty, o_ax,
             x_v, wy_v, y_v, s_v, acc_px, acc_aty, acc_ax,
             rows_v, cols_v, data_v, sem0, sem1, sem2):
        cid = lax.axis_index("c")
        sid = lax.axis_index("s")
        wid = cid * NS + sid
        base = wid * nnz_per_w

        c0 = pltpu.async_copy(x_h, x_v, sem0)
        c1 = pltpu.async_copy(y_h, y_v, sem1)
        c2 = pltpu.async_copy(s_h, s_v, sem2)
        c0.wait()
        c1.wait()
        c2.wait()

        def wy_body(i, _):
            for u in range(4):
                off = (i * 4 + u) * LANES
                sl = pl.ds(off, LANES)
                wy_v[sl] = jnp.maximum(y_v[sl] - s_v[sl], 0.0)
            return 0

        lax.fori_loop(0, m // (4 * LANES), wy_body, 0)

        def zero_body(i, _):
            z = jnp.zeros((LANES,), f32)
            for u in range(4):
                sl = pl.ds((i * 4 + u) * LANES, LANES)
                acc_px[sl] = z
                acc_aty[sl] = z
                acc_ax[sl] = z
            return 0

        lax.fori_loop(0, n // (4 * LANES), zero_body, 0)

        def p_body(g, _):
            off = base + g * CHUNK
            d0 = pltpu.async_copy(p_rows.at[pl.ds(off, CHUNK)], rows_v, sem0)
            d1 = pltpu.async_copy(p_cols.at[pl.ds(off, CHUNK)], cols_v, sem1)
            d2 = pltpu.async_copy(p_data.at[pl.ds(off, CHUNK)], data_v, sem2)
            d0.wait()
            d1.wait()
            d2.wait()
            for i in range(VPC):
                sl = pl.ds(i * LANES, LANES)
                cols = cols_v[sl]
                vals = data_v[sl] * plsc.load_gather(x_v, [cols])
                plsc.addupdate_scatter(acc_px, [rows_v[sl]], vals)
            return 0

        lax.fori_loop(0, n_chunks, p_body, 0)

        def a_body(g, _):
            off = base + g * CHUNK
            d0 = pltpu.async_copy(a_rows.at[pl.ds(off, CHUNK)], rows_v, sem0)
            d1 = pltpu.async_copy(a_cols.at[pl.ds(off, CHUNK)], cols_v, sem1)
            d2 = pltpu.async_copy(a_data.at[pl.ds(off, CHUNK)], data_v, sem2)
            d0.wait()
            d1.wait()
            d2.wait()
            for i in range(VPC):
                sl = pl.ds(i * LANES, LANES)
                rows = rows_v[sl]
                cols = cols_v[sl]
                data = data_v[sl]
                plsc.addupdate_scatter(acc_ax, [rows],
                                       data * plsc.load_gather(x_v, [cols]))
                plsc.addupdate_scatter(acc_aty, [cols],
                                       data * plsc.load_gather(wy_v, [rows]))
            return 0

        lax.fori_loop(0, n_chunks, a_body, 0)

        pltpu.sync_copy(acc_px, o_px.at[wid])
        pltpu.sync_copy(acc_aty, o_aty.at[wid])
        pltpu.sync_copy(acc_ax, o_ax.at[wid])

    return spmv


def _tc_combine(p_px, p_aty, p_ax, q2, b2, x2, y2, s2):
    n = q2.shape[1]
    m = b2.shape[1]
    f32 = jnp.float32

    def body(px_ref, aty_ref, ax_ref, q_ref, b_ref, x_ref, y_ref, s_ref,
             ox_ref, oy_ref, ot_ref):
        px = jnp.sum(px_ref[...], axis=0, keepdims=True)
        aty = jnp.sum(aty_ref[...], axis=0, keepdims=True)
        ax = jnp.sum(ax_ref[...], axis=0, keepdims=True)
        q = q_ref[...]
        b = b_ref[...]
        x = x_ref[...]
        wy = jnp.maximum(y_ref[...] - s_ref[...], 0.0)
        ox_ref[...] = px + aty + q
        oy_ref[...] = b - ax
        xtpx = jnp.sum(x * px)
        qx = jnp.sum(q * x)
        bwy = jnp.sum(b * wy)
        ot_ref[...] = jnp.reshape(-(qx + xtpx) - bwy + 0.5 * xtpx, (1, 1))

    return pl.pallas_call(
        body,
        out_shape=(
            jax.ShapeDtypeStruct((1, n), f32),
            jax.ShapeDtypeStruct((1, m), f32),
            jax.ShapeDtypeStruct((1, 1), f32),
        ),
    )(p_px, p_aty, p_ax, q2, b2, x2, y2, s2)


def kernel(P_data, A_data, q, b, x, y, s, P_rows, P_cols, A_rows, A_cols):
    n = x.shape[0]
    m = y.shape[0]
    nnz = P_data.shape[0]

    spmv = _sc_spmv_kernel(n, m, nnz)
    p_px, p_aty, p_ax = spmv(P_data, P_rows, P_cols, A_data, A_rows, A_cols,
                             x, y, s)

    out_x, out_y, out_t = _tc_combine(
        p_px, p_aty, p_ax,
        q.reshape(1, n), b.reshape(1, m), x.reshape(1, n),
        y.reshape(1, m), s.reshape(1, m))

    return jnp.concatenate(
        [out_x.reshape(-1), out_y.reshape(-1), out_t.reshape(-1)])


# speedup vs baseline: 897.4199x; 2.7183x over previous
import functools

import jax
import jax.numpy as jnp
from jax import lax
from jax.experimental import pallas as pl
from jax.experimental.pallas import tpu as pltpu
from jax.experimental.pallas import tpu_sc as plsc

NC = 2
NS = 16
NW = NC * NS
LANES = 16
CHUNK = 4096
VPC = CHUNK // LANES


def _sc_spmv_kernel(n, m, nnz):
    nnz_per_w = nnz // NW
    n_chunks = nnz_per_w // CHUNK
    assert nnz_per_w % CHUNK == 0 and n_chunks % 2 == 0

    mesh = plsc.VectorSubcoreMesh(core_axis_name="c", subcore_axis_name="s")
    f32 = jnp.float32
    i32 = jnp.int32

    @functools.partial(
        pl.kernel,
        out_type=(
            jax.ShapeDtypeStruct((NW, n), f32),
            jax.ShapeDtypeStruct((NW, n), f32),
            jax.ShapeDtypeStruct((NW, m), f32),
        ),
        mesh=mesh,
        compiler_params=pltpu.CompilerParams(needs_layout_passes=False),
        scratch_types=dict(
            x_v=pltpu.MemorySpace.VMEM((n,), f32),
            wy_v=pltpu.MemorySpace.VMEM((m,), f32),
            acc_px=pltpu.MemorySpace.VMEM((n,), f32),
            acc_aty=pltpu.MemorySpace.VMEM((n,), f32),
            acc_ax=pltpu.MemorySpace.VMEM((m,), f32),
            rows0=pltpu.MemorySpace.VMEM((CHUNK,), i32),
            cols0=pltpu.MemorySpace.VMEM((CHUNK,), i32),
            data0=pltpu.MemorySpace.VMEM((CHUNK,), f32),
            rows1=pltpu.MemorySpace.VMEM((CHUNK,), i32),
            cols1=pltpu.MemorySpace.VMEM((CHUNK,), i32),
            data1=pltpu.MemorySpace.VMEM((CHUNK,), f32),
            tmp_v=pltpu.MemorySpace.VMEM((CHUNK,), f32),
            semr0=pltpu.SemaphoreType.DMA,
            semc0=pltpu.SemaphoreType.DMA,
            semd0=pltpu.SemaphoreType.DMA,
            semr1=pltpu.SemaphoreType.DMA,
            semc1=pltpu.SemaphoreType.DMA,
            semd1=pltpu.SemaphoreType.DMA,
        ),
    )
    def spmv(p_data, p_rows, p_cols, a_data, a_rows, a_cols, x_h, y_h, s_h,
             o_px, o_aty, o_ax,
             x_v, wy_v, acc_px, acc_aty, acc_ax,
             rows0, cols0, data0, rows1, cols1, data1, tmp_v,
             semr0, semc0, semd0, semr1, semc1, semd1):
        cid = lax.axis_index("c")
        sid = lax.axis_index("s")
        wid = cid * NS + sid
        base = wid * nnz_per_w

        sets = ((rows0, cols0, data0, semr0, semc0, semd0),
                (rows1, cols1, data1, semr1, semc1, semd1))

        cx = pltpu.async_copy(x_h, x_v, semr1)
        for ch in range(m // CHUNK):
            off = ch * CHUNK
            cy = pltpu.async_copy(y_h.at[pl.ds(off, CHUNK)], data0, semr0)
            cs = pltpu.async_copy(s_h.at[pl.ds(off, CHUNK)], tmp_v, semc0)
            cy.wait()
            cs.wait()

            @plsc.parallel_loop(0, VPC, unroll=8)
            def _(i):
                sl = pl.ds(i * LANES, LANES)
                wy_v[pl.ds(off + i * LANES, LANES)] = jnp.maximum(
                    data0[sl] - tmp_v[sl], 0.0)

        @plsc.parallel_loop(0, n // LANES, unroll=8)
        def _(i):
            z = jnp.zeros((LANES,), f32)
            sl = pl.ds(i * LANES, LANES)
            acc_px[sl] = z
            acc_aty[sl] = z
            acc_ax[sl] = z

        cx.wait()

        def issue(buf, dh, rh, ch_, off):
            rows_v, cols_v, data_v, sr, sc, sd = buf
            c0 = pltpu.async_copy(rh.at[pl.ds(off, CHUNK)], rows_v, sr)
            c1 = pltpu.async_copy(ch_.at[pl.ds(off, CHUNK)], cols_v, sc)
            c2 = pltpu.async_copy(dh.at[pl.ds(off, CHUNK)], data_v, sd)
            return c0, c1, c2

        def wait(buf, dh, rh, ch_):
            rows_v, cols_v, data_v, sr, sc, sd = buf
            pltpu.make_async_copy(rh.at[pl.ds(0, CHUNK)], rows_v, sr).wait()
            pltpu.make_async_copy(ch_.at[pl.ds(0, CHUNK)], cols_v, sc).wait()
            pltpu.make_async_copy(dh.at[pl.ds(0, CHUNK)], data_v, sd).wait()

        def process_p(buf):
            rows_v, cols_v, data_v, *_ = buf

            @plsc.parallel_loop(0, VPC, unroll=8)
            def _(i):
                sl = pl.ds(i * LANES, LANES)
                vals = data_v[sl] * plsc.load_gather(x_v, [cols_v[sl]])
                plsc.addupdate_scatter(acc_px, [rows_v[sl]], vals)

        def process_a(buf):
            rows_v, cols_v, data_v, *_ = buf

            @plsc.parallel_loop(0, VPC, unroll=8)
            def _(i):
                sl = pl.ds(i * LANES, LANES)
                rows = rows_v[sl]
                cols = cols_v[sl]
                data = data_v[sl]
                plsc.addupdate_scatter(acc_ax, [rows],
                                       data * plsc.load_gather(x_v, [cols]))
                plsc.addupdate_scatter(acc_aty, [cols],
                                       data * plsc.load_gather(wy_v, [rows]))

        def pass_over(dh, rh, ch_, process):
            issue(sets[0], dh, rh, ch_, base)

            def pair_body(p, _):
                off0 = base + (2 * p) * CHUNK
                wait(sets[0], dh, rh, ch_)
                issue(sets[1], dh, rh, ch_, off0 + CHUNK)
                process(sets[0])
                wait(sets[1], dh, rh, ch_)

                @pl.when(2 * p + 2 < n_chunks)
                def _():
                    issue(sets[0], dh, rh, ch_, off0 + 2 * CHUNK)

                process(sets[1])
                return 0

            lax.fori_loop(0, n_chunks // 2, pair_body, 0)

        pass_over(p_data, p_rows, p_cols, process_p)
        pass_over(a_data, a_rows, a_cols, process_a)

        pltpu.sync_copy(acc_px, o_px.at[wid])
        pltpu.sync_copy(acc_aty, o_aty.at[wid])
        pltpu.sync_copy(acc_ax, o_ax.at[wid])

    return spmv


def _tc_combine(p_px, p_aty, p_ax, q2, b2, x2, y2, s2):
    n = q2.shape[1]
    m = b2.shape[1]
    f32 = jnp.float32

    def body(px_ref, aty_ref, ax_ref, q_ref, b_ref, x_ref, y_ref, s_ref,
             ox_ref, oy_ref, ot_ref):
        px = jnp.sum(px_ref[...], axis=0, keepdims=True)
        aty = jnp.sum(aty_ref[...], axis=0, keepdims=True)
        ax = jnp.sum(ax_ref[...], axis=0, keepdims=True)
        q = q_ref[...]
        b = b_ref[...]
        x = x_ref[...]
        wy = jnp.maximum(y_ref[...] - s_ref[...], 0.0)
        ox_ref[...] = px + aty + q
        oy_ref[...] = b - ax
        xtpx = jnp.sum(x * px)
        qx = jnp.sum(q * x)
        bwy = jnp.sum(b * wy)
        ot_ref[...] = jnp.reshape(-(qx + xtpx) - bwy + 0.5 * xtpx, (1, 1))

    return pl.pallas_call(
        body,
        out_shape=(
            jax.ShapeDtypeStruct((1, n), f32),
            jax.ShapeDtypeStruct((1, m), f32),
            jax.ShapeDtypeStruct((1, 1), f32),
        ),
    )(p_px, p_aty, p_ax, q2, b2, x2, y2, s2)


def kernel(P_data, A_data, q, b, x, y, s, P_rows, P_cols, A_rows, A_cols):
    n = x.shape[0]
    m = y.shape[0]
    nnz = P_data.shape[0]

    spmv = _sc_spmv_kernel(n, m, nnz)
    p_px, p_aty, p_ax = spmv(P_data, P_rows, P_cols, A_data, A_rows, A_cols,
                             x, y, s)

    out_x, out_y, out_t = _tc_combine(
        p_px, p_aty, p_ax,
        q.reshape(1, n), b.reshape(1, m), x.reshape(1, n),
        y.reshape(1, m), s.reshape(1, m))

    return jnp.concatenate(
        [out_x.reshape(-1), out_y.reshape(-1), out_t.reshape(-1)])
